# baseline (speedup 1.0000x reference)
import functools

import jax
import jax.numpy as jnp
from jax.experimental import pallas as pl
from jax.experimental.pallas import tpu as pltpu


def _pool_body(x_ref, w_ref, o_ref, acc_ref, *, num_chunks, inv_n):
    j = pl.program_id(0)

    @pl.when(j == 0)
    def _init():
        acc_ref[...] = jnp.zeros_like(acc_ref)

    acc_ref[...] += jnp.sum(x_ref[...], axis=2)

    @pl.when(j == num_chunks - 1)
    def _finish():
        pooled = acc_ref[...] * inv_n
        o_ref[...] = jax.lax.dot_general(
            pooled,
            w_ref[...],
            (((1,), (1,)), ((), ())),
            preferred_element_type=jnp.float32,
        )


def kernel(x, W):
    B, C, H, Wd = x.shape
    N = H * Wd
    E = W.shape[0]
    CHUNK = 4096
    num_chunks = N // CHUNK
    xf = x.reshape(B, C, N)

    grid = (num_chunks,)
    return pl.pallas_call(
        functools.partial(_pool_body, num_chunks=num_chunks, inv_n=1.0 / N),
        grid=grid,
        in_specs=[
            pl.BlockSpec((B, C, CHUNK), lambda j: (0, 0, j)),
            pl.BlockSpec((E, C), lambda j: (0, 0)),
        ],
        out_specs=pl.BlockSpec((B, E), lambda j: (0, 0)),
        out_shape=jax.ShapeDtypeStruct((B, E), jnp.float32),
        scratch_shapes=[pltpu.VMEM((B, C), jnp.float32)],
        compiler_params=pltpu.CompilerParams(
            dimension_semantics=("arbitrary",),
        ),
    )(xf, W)


# speedup vs baseline: 1.3632x; 1.3632x over previous
import functools

import jax
import jax.numpy as jnp
from jax.experimental import pallas as pl
from jax.experimental.pallas import tpu as pltpu


def _rowsum_body(x_ref, o_ref):
    o_ref[...] = jnp.sum(x_ref[...], axis=1, keepdims=True)


def _proj_body(p_ref, w_ref, o_ref, *, inv_n):
    pooled = p_ref[...] * inv_n
    o_ref[...] = jax.lax.dot_general(
        pooled,
        w_ref[...],
        (((1,), (1,)), ((), ())),
        preferred_element_type=jnp.float32,
    )


def kernel(x, W):
    B, C, H, Wd = x.shape
    N = H * Wd
    E = W.shape[0]
    R = B * C
    ROWS = 8
    num_steps = R // ROWS

    xf = x.reshape(R, N)

    rowsums = pl.pallas_call(
        _rowsum_body,
        grid=(num_steps,),
        in_specs=[pl.BlockSpec((ROWS, N), lambda i: (i, 0))],
        out_specs=pl.BlockSpec((ROWS, 1), lambda i: (i, 0)),
        out_shape=jax.ShapeDtypeStruct((R, 1), jnp.float32),
        compiler_params=pltpu.CompilerParams(
            dimension_semantics=("arbitrary",),
        ),
    )(xf)

    pooled = rowsums.reshape(B, C)

    return pl.pallas_call(
        functools.partial(_proj_body, inv_n=1.0 / N),
        in_specs=[
            pl.BlockSpec((B, C), lambda: (0, 0)),
            pl.BlockSpec((E, C), lambda: (0, 0)),
        ],
        out_specs=pl.BlockSpec((B, E), lambda: (0, 0)),
        out_shape=jax.ShapeDtypeStruct((B, E), jnp.float32),
    )(pooled, W)


# speedup vs baseline: 1.4320x; 1.0504x over previous
import functools

import jax
import jax.numpy as jnp
from jax.experimental import pallas as pl
from jax.experimental.pallas import tpu as pltpu


def _rowsum_body(x_ref, o_ref):
    o_ref[...] = jnp.sum(x_ref[...], axis=1, keepdims=True)


def _proj_body(p_ref, w_ref, o_ref, *, inv_n):
    pooled = p_ref[...] * inv_n
    o_ref[...] = jax.lax.dot_general(
        pooled,
        w_ref[...],
        (((1,), (1,)), ((), ())),
        preferred_element_type=jnp.float32,
    )


def kernel(x, W):
    B, C, H, Wd = x.shape
    N = H * Wd
    E = W.shape[0]
    R = B * C
    ROWS = 8
    num_steps = R // ROWS

    xf = x.reshape(R, N)

    rowsums = pl.pallas_call(
        _rowsum_body,
        grid=(num_steps,),
        in_specs=[pl.BlockSpec((ROWS, N), lambda i: (i, 0))],
        out_specs=pl.BlockSpec((ROWS, 1), lambda i: (i, 0)),
        out_shape=jax.ShapeDtypeStruct((R, 1), jnp.float32),
        compiler_params=pltpu.CompilerParams(
            dimension_semantics=("parallel",),
        ),
    )(xf)

    pooled = rowsums.reshape(B, C)

    return pl.pallas_call(
        functools.partial(_proj_body, inv_n=1.0 / N),
        in_specs=[
            pl.BlockSpec((B, C), lambda: (0, 0)),
            pl.BlockSpec((E, C), lambda: (0, 0)),
        ],
        out_specs=pl.BlockSpec((B, E), lambda: (0, 0)),
        out_shape=jax.ShapeDtypeStruct((B, E), jnp.float32),
    )(pooled, W)
